# Initial kernel scaffold; baseline (speedup 1.0000x reference)
import jax
import jax.numpy as jnp
from jax.experimental import pallas as pl


def kernel(t, x, embed_table, wq, bq, wk, bk):
    raise NotImplementedError("write your pallas kernel here")


# speedup vs baseline: 32.5656x; 32.5656x over previous
import jax
import jax.numpy as jnp
from jax.experimental import pallas as pl

_DATA_DIM = 2048
_QK_SCALE = 16 ** -0.5
_SUBSTEPS = 8


def _ode_kernel(h_ref, x_ref, ep_ref, wq_ref, bq_ref, wk_ref, bk_ref, out_ref):
    x0 = x_ref[...]
    ep = ep_ref[...]
    wq = wq_ref[...]
    wk = wk_ref[...]
    qe = jnp.dot(ep, wq, preferred_element_type=jnp.float32) + bq_ref[...]
    ke = jnp.dot(ep, wk, preferred_element_type=jnp.float32) + bk_ref[...]
    wq0 = wq[0:1, :]
    wk0 = wk[0:1, :]
    h = h_ref[0, 0]

    def rhs(x):
        ss = jnp.sum(x * x, axis=1, keepdims=True)
        u = jnp.dot(x, ke, preferred_element_type=jnp.float32)
        m = ss * wk0 + u
        a = jnp.sum(m * wq0, axis=1, keepdims=True)
        proj = jax.lax.dot_general(m, qe, (((1,), (1,)), ((), ())),
                                   preferred_element_type=jnp.float32)
        fit = _QK_SCALE * (a * x + proj)
        g = jnp.sum(x * fit, axis=1, keepdims=True)
        return x * (fit - g)

    def step(_, x):
        k1 = rhs(x)
        k2 = rhs(x + (0.5 * h) * k1)
        k3 = rhs(x + (0.5 * h) * k2)
        k4 = rhs(x + h * k3)
        return x + (h / 6.0) * (k1 + 2.0 * k2 + 2.0 * k3 + k4)

    xf = jax.lax.fori_loop(0, _SUBSTEPS, step, x0)
    out_ref[0, :, :] = x0
    out_ref[1, :, :] = xf


def kernel(t, x, embed_table, wq, bq, wk, bk):
    B, D = x.shape
    ep = jnp.concatenate(
        [jnp.zeros((D, 1), jnp.float32), embed_table[1:D + 1]], axis=1)
    h = ((t[1] - t[0]) / _SUBSTEPS).reshape(1, 1)
    out = pl.pallas_call(
        _ode_kernel,
        out_shape=jax.ShapeDtypeStruct((2, B, D), jnp.float32),
    )(h, x, ep, wq, bq.reshape(1, -1), wk, bk.reshape(1, -1))
    return out
